# baseline (speedup 1.0000x reference)
import jax
import jax.numpy as jnp
from jax.experimental import pallas as pl
from jax.experimental.pallas import tpu as pltpu

E = 64
K = 8
ALPHA = 0.01
ROWS = 512


def _gate_kernel(x_ref, w_ref, idx_ref, val_ref, acc_ref):
    x = x_ref[...]
    w = w_ref[...]
    logits = jax.lax.dot_general(
        w, x, (((1,), (1,)), ((), ())), preferred_element_type=jnp.float32
    )
    m = jnp.max(logits, axis=0, keepdims=True)
    e = jnp.exp(logits - m)
    denom = jnp.sum(e, axis=0, keepdims=True)
    scores = e / denom

    acc_ref[...] = jnp.full((1, 8, 128), jnp.sum(scores), jnp.float32)

    iota = jax.lax.broadcasted_iota(jnp.int32, scores.shape, 0)
    work = scores
    vals = []
    idxs = []
    for _ in range(K):
        mk = jnp.max(work, axis=0, keepdims=True)
        sel = jnp.min(jnp.where(work == mk, iota, E), axis=0, keepdims=True)
        vals.append(mk)
        idxs.append(sel)
        work = jnp.where(iota == sel, -1.0, work)
    val_ref[...] = jnp.concatenate(vals, axis=0)
    idx_ref[...] = jnp.concatenate(idxs, axis=0)


def kernel(hidden_states, weight):
    b, s, h = hidden_states.shape
    n = b * s
    hs = hidden_states.reshape(n, h)
    nblk = n // ROWS
    idx_t, val_t, acc = pl.pallas_call(
        _gate_kernel,
        grid=(nblk,),
        in_specs=[
            pl.BlockSpec((ROWS, h), lambda i: (i, 0)),
            pl.BlockSpec((E, h), lambda i: (0, 0)),
        ],
        out_specs=[
            pl.BlockSpec((K, ROWS), lambda i: (0, i)),
            pl.BlockSpec((K, ROWS), lambda i: (0, i)),
            pl.BlockSpec((1, 8, 128), lambda i: (i, 0, 0)),
        ],
        out_shape=[
            jax.ShapeDtypeStruct((K, n), jnp.int32),
            jax.ShapeDtypeStruct((K, n), jnp.float32),
            jax.ShapeDtypeStruct((nblk, 8, 128), jnp.float32),
        ],
        compiler_params=pltpu.CompilerParams(
            dimension_semantics=("parallel",)
        ),
    )(hs, weight)
    aux_loss = jnp.sum(acc[:, 0, 0]) * (ALPHA / n)
    return idx_t.T, val_t.T, aux_loss


# speedup vs baseline: 1.2237x; 1.2237x over previous
import jax
import jax.numpy as jnp
from jax.experimental import pallas as pl
from jax.experimental.pallas import tpu as pltpu

E = 64
K = 8
ALPHA = 0.01
ROWS = 1024


def _gate_kernel(x_ref, w_ref, idx_ref, val_ref, acc_ref):
    x = x_ref[...]
    w = w_ref[...]
    logits = jax.lax.dot_general(
        w, x, (((1,), (1,)), ((), ())), preferred_element_type=jnp.float32
    )
    m = jnp.max(logits, axis=0, keepdims=True)
    e = jnp.exp(logits - m)
    denom = jnp.sum(e, axis=0, keepdims=True)
    scores = e / denom

    @pl.when(pl.program_id(0) == 0)
    def _init():
        acc_ref[0, 0] = 0.0

    acc_ref[0, 0] += jnp.sum(scores)

    iota = jax.lax.broadcasted_iota(jnp.int32, scores.shape, 0)
    work = scores
    vals = []
    idxs = []
    for _ in range(K):
        mk = jnp.max(work, axis=0, keepdims=True)
        sel = jnp.min(jnp.where(work == mk, iota, E), axis=0, keepdims=True)
        vals.append(mk)
        idxs.append(sel)
        work = jnp.where(iota == sel, -1.0, work)
    val_ref[...] = jnp.concatenate(vals, axis=0)
    idx_ref[...] = jnp.concatenate(idxs, axis=0)


def kernel(hidden_states, weight):
    b, s, h = hidden_states.shape
    n = b * s
    hs = hidden_states.reshape(n, h)
    nblk = n // ROWS
    idx_t, val_t, acc = pl.pallas_call(
        _gate_kernel,
        grid=(nblk,),
        in_specs=[
            pl.BlockSpec((ROWS, h), lambda i: (i, 0)),
            pl.BlockSpec((E, h), lambda i: (0, 0)),
        ],
        out_specs=[
            pl.BlockSpec((K, ROWS), lambda i: (0, i)),
            pl.BlockSpec((K, ROWS), lambda i: (0, i)),
            pl.BlockSpec(memory_space=pltpu.SMEM),
        ],
        out_shape=[
            jax.ShapeDtypeStruct((K, n), jnp.int32),
            jax.ShapeDtypeStruct((K, n), jnp.float32),
            jax.ShapeDtypeStruct((1, 1), jnp.float32),
        ],
    )(hs, weight)
    aux_loss = acc[0, 0] * (ALPHA / n)
    return idx_t.T, val_t.T, aux_loss
